# baseline (speedup 1.0000x reference)
import functools

import jax
import jax.numpy as jnp
from jax import lax
from jax.experimental import pallas as pl
from jax.experimental.pallas import tpu as pltpu
from jax.experimental.pallas import tpu_sc as plsc

N = 50000
D = 128
DOUT = 64
B = 16384
S = 16
EPS = 1e-5

NC = 2
NS = 16
NW = NC * NS
INV_S = 1.0 / S

CHUNK = 64
NSPLIT = 2


def _make_sc_gather(bsz):
    bpw = bsz // NW
    nacc = bpw // CHUNK

    def _sc_body(x0_hbm, h1_hbm, nidxt_hbm, nb_hbm, agg_hbm, hb_hbm,
                 nidxt_v, nb_v, acc_v, sems, hsem):
        wid = lax.axis_index("s") * NC + lax.axis_index("c")
        bbase = wid * bpw
        pltpu.sync_copy(nidxt_hbm.at[pl.ds(wid * bpw * S, bpw * S)], nidxt_v)
        pltpu.sync_copy(nb_hbm.at[pl.ds(bbase, bpw)], nb_v)

        def acc_dst(c):
            return acc_v.at[pl.ds(c * CHUNK, CHUNK)]

        for s_ in range(S):
            for c in range(nacc):
                idx = nidxt_v.at[pl.ds(s_ * bpw + c * CHUNK, CHUNK)]
                if s_ > 0:
                    pltpu.make_async_copy(
                        x0_hbm.at[idx], acc_dst(c), sems.at[c]).wait()
                    pltpu.async_copy(x0_hbm.at[idx], acc_dst(c), sems.at[c],
                                     add=True)
                else:
                    pltpu.async_copy(x0_hbm.at[idx], acc_dst(c), sems.at[c])

        for c in range(nacc):
            pltpu.make_async_copy(
                x0_hbm.at[nidxt_v.at[pl.ds(0, CHUNK)]], acc_dst(c), sems.at[c]
            ).wait()
            pltpu.sync_copy(acc_dst(c),
                            agg_hbm.at[pl.ds(bbase + c * CHUNK, CHUNK)])
            pltpu.async_copy(h1_hbm.at[nb_v.at[pl.ds(c * CHUNK, CHUNK)]],
                             acc_dst(c), hsem)

        for c in range(nacc):
            pltpu.make_async_copy(
                h1_hbm.at[nb_v.at[pl.ds(0, CHUNK)]], acc_dst(c), hsem).wait()
        pltpu.sync_copy(acc_v, hb_hbm.at[pl.ds(bbase, bpw)])

    return functools.partial(
        pl.kernel,
        out_type=[
            jax.ShapeDtypeStruct((bsz, D), jnp.float32),
            jax.ShapeDtypeStruct((bsz, D), jnp.float32),
        ],
        mesh=plsc.VectorSubcoreMesh(core_axis_name="c", subcore_axis_name="s"),
        scratch_types=[
            pltpu.VMEM((bpw * S,), jnp.int32),
            pltpu.VMEM((bpw,), jnp.int32),
            pltpu.VMEM((bpw, D), jnp.float32),
            pltpu.SemaphoreType.DMA((nacc,)),
            pltpu.SemaphoreType.DMA,
        ],
    )(_sc_body)


def _tc_body(agg_ref, hb_ref, w2a_ref, w2b_ref, b2_ref, g2_ref, be2_ref,
             wout_ref, bout_ref, out_ref):
    h = jnp.dot(agg_ref[...], w2a_ref[...],
                preferred_element_type=jnp.float32) * jnp.float32(INV_S)
    h = h + jnp.dot(hb_ref[...], w2b_ref[...], preferred_element_type=jnp.float32)
    h = h + b2_ref[...]
    h = jnp.maximum(h, 0.0)
    mu = jnp.mean(h, axis=1, keepdims=True)
    d = h - mu
    var = jnp.mean(d * d, axis=1, keepdims=True)
    h = d * lax.rsqrt(var + EPS) * g2_ref[...] + be2_ref[...]
    logits = jnp.dot(h, wout_ref[...], preferred_element_type=jnp.float32)
    logits = logits + bout_ref[...]
    m = jnp.max(logits, axis=1, keepdims=True)
    e = jnp.exp(logits - m)
    out_ref[...] = e / jnp.sum(e, axis=1, keepdims=True)


TC_BLK = 4096


def _tc_dense(agg, hb, w2a, w2b, b2, g2, be2, wout, bout):
    bsz = agg.shape[0]
    blk = min(TC_BLK, bsz)
    grid = (bsz // blk,)
    row_blk = pl.BlockSpec((blk, D), lambda i: (i, 0))

    def rep(shape):
        return pl.BlockSpec(shape, lambda i: (0, 0))

    return pl.pallas_call(
        _tc_body,
        grid=grid,
        in_specs=[
            row_blk,
            row_blk,
            rep((D, D)),
            rep((D, D)),
            rep((1, D)),
            rep((1, D)),
            rep((1, D)),
            rep((D, DOUT)),
            rep((1, DOUT)),
        ],
        out_specs=pl.BlockSpec((blk, DOUT), lambda i: (i, 0)),
        out_shape=jax.ShapeDtypeStruct((bsz, DOUT), jnp.float32),
    )(agg, hb, w2a, w2b, b2, g2, be2, wout, bout)


_sc_gather_slice = _make_sc_gather(B // NSPLIT)


def kernel(x0, h1, node_batch, neigh_idx_1, neigh_idx_2,
           W1, b1, g1, be1, W2, b2, g2, be2, Wout, bout):
    del neigh_idx_1, W1, b1, g1, be1
    bsz = B // NSPLIT
    bpw = bsz // NW
    nidxt = (neigh_idx_2.astype(jnp.int32)
             .reshape(NSPLIT, NW, bpw, S).transpose(0, 1, 3, 2)
             .reshape(NSPLIT, -1))
    nb = node_batch.astype(jnp.int32).reshape(NSPLIT, bsz)
    w2a, w2b = W2[:D], W2[D:]
    b2r, g2r, be2r = b2.reshape(1, D), g2.reshape(1, D), be2.reshape(1, D)
    boutr = bout.reshape(1, DOUT)
    outs = []
    for k in range(NSPLIT):
        agg, hb = _sc_gather_slice(x0, h1, nidxt[k], nb[k])
        outs.append(_tc_dense(agg, hb, w2a, w2b, b2r, g2r, be2r, Wout, boutr))
    return jnp.concatenate(outs, axis=0)


# speedup vs baseline: 1.2095x; 1.2095x over previous
import functools

import jax
import jax.numpy as jnp
from jax import lax
from jax.experimental import pallas as pl
from jax.experimental.pallas import tpu as pltpu
from jax.experimental.pallas import tpu_sc as plsc

N = 50000
D = 128
DOUT = 64
B = 16384
S = 16
EPS = 1e-5

NC = 2
NS = 16
NW = NC * NS
INV_S = 1.0 / S

CHUNK = 64
NSPLIT = 1


def _make_sc_gather(bsz):
    bpw = bsz // NW
    nacc = bpw // CHUNK

    def _sc_body(x0_hbm, h1_hbm, nidxt_hbm, nb_hbm, agg_hbm, hb_hbm,
                 nidxt_v, nb_v, acc_v, sems, hsem):
        wid = lax.axis_index("s") * NC + lax.axis_index("c")
        bbase = wid * bpw
        pltpu.sync_copy(nidxt_hbm.at[pl.ds(wid * bpw * S, bpw * S)], nidxt_v)
        pltpu.sync_copy(nb_hbm.at[pl.ds(bbase, bpw)], nb_v)

        def acc_dst(c):
            return acc_v.at[pl.ds(c * CHUNK, CHUNK)]

        for s_ in range(S):
            for c in range(nacc):
                idx = nidxt_v.at[pl.ds(s_ * bpw + c * CHUNK, CHUNK)]
                if s_ > 0:
                    pltpu.make_async_copy(
                        x0_hbm.at[idx], acc_dst(c), sems.at[c]).wait()
                    pltpu.async_copy(x0_hbm.at[idx], acc_dst(c), sems.at[c],
                                     add=True)
                else:
                    pltpu.async_copy(x0_hbm.at[idx], acc_dst(c), sems.at[c])

        for c in range(nacc):
            pltpu.make_async_copy(
                x0_hbm.at[nidxt_v.at[pl.ds(0, CHUNK)]], acc_dst(c), sems.at[c]
            ).wait()
            pltpu.sync_copy(acc_dst(c),
                            agg_hbm.at[pl.ds(bbase + c * CHUNK, CHUNK)])
            pltpu.async_copy(h1_hbm.at[nb_v.at[pl.ds(c * CHUNK, CHUNK)]],
                             acc_dst(c), hsem)

        for c in range(nacc):
            pltpu.make_async_copy(
                h1_hbm.at[nb_v.at[pl.ds(0, CHUNK)]], acc_dst(c), hsem).wait()
        pltpu.sync_copy(acc_v, hb_hbm.at[pl.ds(bbase, bpw)])

    return functools.partial(
        pl.kernel,
        out_type=[
            jax.ShapeDtypeStruct((bsz, D), jnp.float32),
            jax.ShapeDtypeStruct((bsz, D), jnp.float32),
        ],
        mesh=plsc.VectorSubcoreMesh(core_axis_name="c", subcore_axis_name="s"),
        scratch_types=[
            pltpu.VMEM((bpw * S,), jnp.int32),
            pltpu.VMEM((bpw,), jnp.int32),
            pltpu.VMEM((bpw, D), jnp.float32),
            pltpu.SemaphoreType.DMA((nacc,)),
            pltpu.SemaphoreType.DMA,
        ],
    )(_sc_body)


def _tc_body(agg_ref, hb_ref, w2a_ref, w2b_ref, b2_ref, g2_ref, be2_ref,
             wout_ref, bout_ref, out_ref):
    h = jnp.dot(agg_ref[...], w2a_ref[...],
                preferred_element_type=jnp.float32) * jnp.float32(INV_S)
    h = h + jnp.dot(hb_ref[...], w2b_ref[...], preferred_element_type=jnp.float32)
    h = h + b2_ref[...]
    h = jnp.maximum(h, 0.0)
    mu = jnp.mean(h, axis=1, keepdims=True)
    d = h - mu
    var = jnp.mean(d * d, axis=1, keepdims=True)
    h = d * lax.rsqrt(var + EPS) * g2_ref[...] + be2_ref[...]
    logits = jnp.dot(h, wout_ref[...], preferred_element_type=jnp.float32)
    logits = logits + bout_ref[...]
    m = jnp.max(logits, axis=1, keepdims=True)
    e = jnp.exp(logits - m)
    out_ref[...] = e / jnp.sum(e, axis=1, keepdims=True)


TC_BLK = 4096


def _tc_dense(agg, hb, w2a, w2b, b2, g2, be2, wout, bout):
    bsz = agg.shape[0]
    blk = min(TC_BLK, bsz)
    grid = (bsz // blk,)
    row_blk = pl.BlockSpec((blk, D), lambda i: (i, 0))

    def rep(shape):
        return pl.BlockSpec(shape, lambda i: (0, 0))

    return pl.pallas_call(
        _tc_body,
        grid=grid,
        in_specs=[
            row_blk,
            row_blk,
            rep((D, D)),
            rep((D, D)),
            rep((1, D)),
            rep((1, D)),
            rep((1, D)),
            rep((D, DOUT)),
            rep((1, DOUT)),
        ],
        out_specs=pl.BlockSpec((blk, DOUT), lambda i: (i, 0)),
        out_shape=jax.ShapeDtypeStruct((bsz, DOUT), jnp.float32),
    )(agg, hb, w2a, w2b, b2, g2, be2, wout, bout)


_sc_gather_slice = _make_sc_gather(B // NSPLIT)


def kernel(x0, h1, node_batch, neigh_idx_1, neigh_idx_2,
           W1, b1, g1, be1, W2, b2, g2, be2, Wout, bout):
    del neigh_idx_1, W1, b1, g1, be1
    bsz = B // NSPLIT
    bpw = bsz // NW
    nidxt = (neigh_idx_2.astype(jnp.int32)
             .reshape(NSPLIT, NW, bpw, S).transpose(0, 1, 3, 2)
             .reshape(NSPLIT, -1))
    nb = node_batch.astype(jnp.int32).reshape(NSPLIT, bsz)
    w2a, w2b = W2[:D], W2[D:]
    b2r, g2r, be2r = b2.reshape(1, D), g2.reshape(1, D), be2.reshape(1, D)
    boutr = bout.reshape(1, DOUT)
    outs = []
    for k in range(NSPLIT):
        agg, hb = _sc_gather_slice(x0, h1, nidxt[k], nb[k])
        outs.append(_tc_dense(agg, hb, w2a, w2b, b2r, g2r, be2r, Wout, boutr))
    return jnp.concatenate(outs, axis=0)
